# baseline (speedup 1.0000x reference)
import jax
import jax.numpy as jnp
from jax.experimental import pallas as pl
from jax.experimental.pallas import tpu as pltpu

_WINDOW = 100


def _focal_body(alpha_ref, x_ref, t_ref, k_ref, p_ref,
                num_m_ref, cnt_m_ref, num_a_ref, pos_ref):
    z = x_ref[0]
    t = t_ref[0]

    hi = z.astype(jnp.bfloat16)
    lo = (z - hi.astype(jnp.float32)).astype(jnp.bfloat16)
    perm = p_ref[...]
    xg = (jnp.dot(hi, perm, preferred_element_type=jnp.float32)
          + jnp.dot(lo, perm, preferred_element_type=jnp.float32))
    x0 = xg[:, 0:128]
    x1 = xg[:, 128:256]
    x2 = xg[:, 256:384]

    m = jnp.maximum(jnp.maximum(x0, x1), x2)
    e0 = jnp.exp(x0 - m)
    e1 = jnp.exp(x1 - m)
    e2 = jnp.exp(x2 - m)
    se = e0 + e1 + e2

    c1 = t == 1
    c2 = t == 2
    xt = jnp.where(c2, x2, jnp.where(c1, x1, x0))
    et = jnp.where(c2, e2, jnp.where(c1, e1, e0))
    ce = jnp.log(se) - (xt - m)
    pt = et * (1.0 / se)

    a0 = alpha_ref[0]
    a1 = alpha_ref[1]
    a2 = alpha_ref[2]
    at = jnp.where(c2, a2, jnp.where(c1, a1, a0))
    om = 1.0 - pt
    focal = at * (om * om) * ce

    posf = jnp.where(t > 0, 1.0, 0.0)
    pb = posf.astype(jnp.bfloat16)
    r_rows = pb.shape[0]
    zrow = jnp.zeros((1, 128), jnp.bfloat16)
    p_prev = jnp.concatenate([zrow, pb[:r_rows - 1]], axis=0)
    p_next = jnp.concatenate([pb[1:], zrow], axis=0)
    cnt = (jnp.dot(p_prev, k_ref[0], preferred_element_type=jnp.float32)
           + jnp.dot(pb, k_ref[1], preferred_element_type=jnp.float32)
           + jnp.dot(p_next, k_ref[2], preferred_element_type=jnp.float32))
    mf = jnp.where(cnt > 0.5, 1.0, 0.0)

    num_m_ref[0] = jnp.sum(focal * mf, axis=0, keepdims=True)
    cnt_m_ref[0] = jnp.sum(mf, axis=0, keepdims=True)
    num_a_ref[0] = jnp.sum(focal, axis=0, keepdims=True)
    pos_ref[0] = jnp.sum(posf, axis=0, keepdims=True)


@jax.jit
def kernel(inputs, targets, alpha):
    B, S, C = inputs.shape
    R = S // 128
    x = inputs.reshape(B, R, 128 * C)
    t4 = targets.reshape(B, R, 128)

    q = jax.lax.broadcasted_iota(jnp.int32, (128, 128), 0)
    p = jax.lax.broadcasted_iota(jnp.int32, (128, 128), 1)
    d = q - p
    k_prev = d >= 128 - _WINDOW
    k_cur = jnp.abs(d) <= _WINDOW
    k_next = d <= _WINDOW - 128
    kmats = jnp.stack([k_prev, k_cur, k_next]).astype(jnp.bfloat16)

    row = jax.lax.broadcasted_iota(jnp.int32, (128 * C, 128 * C), 0)
    col = jax.lax.broadcasted_iota(jnp.int32, (128 * C, 128 * C), 1)
    perm = ((row % C) * 128 + row // C == col).astype(jnp.bfloat16)

    outs = pl.pallas_call(
        _focal_body,
        grid=(B,),
        in_specs=[
            pl.BlockSpec(memory_space=pltpu.SMEM),
            pl.BlockSpec((1, R, 128 * C), lambda b: (b, 0, 0)),
            pl.BlockSpec((1, R, 128), lambda b: (b, 0, 0)),
            pl.BlockSpec((3, 128, 128), lambda b: (0, 0, 0)),
            pl.BlockSpec((128 * C, 128 * C), lambda b: (0, 0)),
        ],
        out_specs=[pl.BlockSpec((1, 1, 128), lambda b: (b, 0, 0))] * 4,
        out_shape=[jax.ShapeDtypeStruct((B, 1, 128), jnp.float32)] * 4,
        compiler_params=pltpu.CompilerParams(
            dimension_semantics=("parallel",),
        ),
        name="masked_focal_loss",
    )(alpha, x, t4, kmats, perm)

    num_m, cnt_m, num_a, posc = [o.sum(axis=(1, 2)) for o in outs]
    has_pos = posc > 0
    num = jnp.where(has_pos, num_m, num_a)
    den = jnp.where(has_pos, cnt_m, jnp.float32(S))
    return jnp.sum(num) / jnp.sum(den)


# speedup vs baseline: 6.8052x; 6.8052x over previous
import jax
import jax.numpy as jnp
from jax.experimental import pallas as pl
from jax.experimental.pallas import tpu as pltpu

_WINDOW = 100
_WLEN = 2 * _WINDOW + 1


def _focal_body(alpha_ref, x_ref, t_ref, hp_ref, hn_ref,
                num_m_ref, cnt_m_ref, num_a_ref, pos_ref, *, n_j):
    j = pl.program_id(1)
    x0 = x_ref[0]
    x1 = x_ref[1]
    x2 = x_ref[2]
    t = t_ref[...]
    sb = t.shape[1]

    m = jnp.maximum(jnp.maximum(x0, x1), x2)
    e0 = jnp.exp(x0 - m)
    e1 = jnp.exp(x1 - m)
    e2 = jnp.exp(x2 - m)
    se = e0 + e1 + e2

    c1 = t == 1
    c2 = t == 2
    xt = jnp.where(c2, x2, jnp.where(c1, x1, x0))
    et = jnp.where(c2, e2, jnp.where(c1, e1, e0))
    ce = jnp.log(se) - (xt - m)
    pt = et * (1.0 / se)

    a0 = alpha_ref[0]
    a1 = alpha_ref[1]
    a2 = alpha_ref[2]
    at = jnp.where(c2, a2, jnp.where(c1, a1, a0))
    om = 1.0 - pt
    focal = at * (om * om) * ce

    pos = jnp.where(t > 0, 1.0, 0.0)
    hp = jnp.where((hp_ref[...] > 0) & (j > 0), 1.0, 0.0)
    hn = jnp.where((hn_ref[...] > 0) & (j < n_j - 1), 1.0, 0.0)
    f = jnp.concatenate([hp, pos, hn], axis=1)
    n = 1
    while n < _WLEN:
        s = min(n, _WLEN - n)
        f = jnp.maximum(f, jnp.concatenate(
            [f[:, s:], jnp.zeros((8, s), jnp.float32)], axis=1))
        n += s
    mf = f[:, 128 - _WINDOW:128 - _WINDOW + sb]

    ones = jnp.ones((8, 128), jnp.float32)
    nm = jnp.sum(focal * mf, axis=1, keepdims=True) * ones
    cm = jnp.sum(mf, axis=1, keepdims=True) * ones
    na = jnp.sum(focal, axis=1, keepdims=True) * ones
    pc = jnp.sum(pos, axis=1, keepdims=True) * ones

    @pl.when(j == 0)
    def _():
        num_m_ref[...] = nm
        cnt_m_ref[...] = cm
        num_a_ref[...] = na
        pos_ref[...] = pc

    @pl.when(j > 0)
    def _():
        num_m_ref[...] += nm
        cnt_m_ref[...] += cm
        num_a_ref[...] += na
        pos_ref[...] += pc


@jax.jit
def kernel(inputs, targets, alpha):
    import functools
    B, S, C = inputs.shape
    x = jnp.transpose(inputs, (2, 0, 1))
    SB = 16384 if S % 16384 == 0 else S
    KB = SB // 128
    NJ = S // SB
    KMAX = S // 128 - 1

    outs = pl.pallas_call(
        functools.partial(_focal_body, n_j=NJ),
        grid=(B // 8, NJ),
        in_specs=[
            pl.BlockSpec(memory_space=pltpu.SMEM),
            pl.BlockSpec((C, 8, SB), lambda i, j: (0, i, j)),
            pl.BlockSpec((8, SB), lambda i, j: (i, j)),
            pl.BlockSpec((8, 128), lambda i, j: (i, jnp.maximum(j * KB - 1, 0))),
            pl.BlockSpec((8, 128), lambda i, j: (i, jnp.minimum((j + 1) * KB, KMAX))),
        ],
        out_specs=[pl.BlockSpec((8, 128), lambda i, j: (i, 0))] * 4,
        out_shape=[jax.ShapeDtypeStruct((B, 128), jnp.float32)] * 4,
        compiler_params=pltpu.CompilerParams(
            dimension_semantics=("parallel", "arbitrary"),
        ),
        name="masked_focal_loss",
    )(alpha, x, targets, targets, targets)

    num_m, cnt_m, num_a, posc = [o[:, 0] for o in outs]
    has_pos = posc > 0
    num = jnp.where(has_pos, num_m, num_a)
    den = jnp.where(has_pos, cnt_m, jnp.float32(S))
    return jnp.sum(num) / jnp.sum(den)


# speedup vs baseline: 7.0514x; 1.0362x over previous
import functools

import jax
import jax.numpy as jnp
from jax.experimental import pallas as pl
from jax.experimental.pallas import tpu as pltpu

_WINDOW = 100
_WLEN = 2 * _WINDOW + 1


def _focal_body(alpha_ref, x_ref, t_ref, hp_ref, hn_ref,
                num_m_ref, cnt_m_ref, num_a_ref, pos_ref, *, n_j):
    j = pl.program_id(1)
    x0 = x_ref[0]
    x1 = x_ref[1]
    x2 = x_ref[2]
    t = t_ref[...]
    sb = t.shape[1]

    d1 = x1 - x0
    d2 = x2 - x0
    e1 = jnp.exp(d1)
    e2 = jnp.exp(d2)
    se = 1.0 + e1 + e2

    c1 = t == 1
    c2 = t == 2
    dt = jnp.where(c2, d2, jnp.where(c1, d1, 0.0))
    et = jnp.where(c2, e2, jnp.where(c1, e1, 1.0))
    ce = jnp.log(se) - dt
    pt = et * (1.0 / se)

    a0 = alpha_ref[0]
    a1 = alpha_ref[1]
    a2 = alpha_ref[2]
    at = jnp.where(c2, a2, jnp.where(c1, a1, a0))
    om = 1.0 - pt
    focal = at * (om * om) * ce

    pos = jnp.where(t > 0, 1.0, 0.0)
    hp = jnp.where((hp_ref[...] > 0) & (j > 0), 1.0, 0.0)
    hn = jnp.where((hn_ref[...] > 0) & (j < n_j - 1), 1.0, 0.0)
    f = jnp.concatenate([hp, pos, hn], axis=1)
    n = 1
    while n < _WLEN:
        s = min(n, _WLEN - n)
        f = jnp.maximum(f, jnp.concatenate(
            [f[:, s:], jnp.zeros((8, s), jnp.float32)], axis=1))
        n += s
    mf = f[:, 128 - _WINDOW:128 - _WINDOW + sb]

    ones = jnp.ones((8, 128), jnp.float32)
    nm = jnp.sum(focal * mf, axis=1, keepdims=True) * ones
    cm = jnp.sum(mf, axis=1, keepdims=True) * ones
    na = jnp.sum(focal, axis=1, keepdims=True) * ones
    pc = jnp.sum(pos, axis=1, keepdims=True) * ones

    @pl.when(j == 0)
    def _():
        num_m_ref[...] = nm
        cnt_m_ref[...] = cm
        num_a_ref[...] = na
        pos_ref[...] = pc

    @pl.when(j > 0)
    def _():
        num_m_ref[...] += nm
        cnt_m_ref[...] += cm
        num_a_ref[...] += na
        pos_ref[...] += pc


@jax.jit
def kernel(inputs, targets, alpha):
    B, S, C = inputs.shape
    x = jnp.transpose(inputs, (2, 0, 1))
    SB = 16384 if S % 16384 == 0 else S
    KB = SB // 128
    NJ = S // SB
    KMAX = S // 128 - 1

    outs = pl.pallas_call(
        functools.partial(_focal_body, n_j=NJ),
        grid=(B // 8, NJ),
        in_specs=[
            pl.BlockSpec(memory_space=pltpu.SMEM),
            pl.BlockSpec((C, 8, SB), lambda i, j: (0, i, j)),
            pl.BlockSpec((8, SB), lambda i, j: (i, j)),
            pl.BlockSpec((8, 128), lambda i, j: (i, jnp.maximum(j * KB - 1, 0))),
            pl.BlockSpec((8, 128), lambda i, j: (i, jnp.minimum((j + 1) * KB, KMAX))),
        ],
        out_specs=[pl.BlockSpec((8, 128), lambda i, j: (i, 0))] * 4,
        out_shape=[jax.ShapeDtypeStruct((B, 128), jnp.float32)] * 4,
        compiler_params=pltpu.CompilerParams(
            dimension_semantics=("parallel", "arbitrary"),
        ),
        name="masked_focal_loss",
    )(alpha, x, targets, targets, targets)

    num_m, cnt_m, num_a, posc = [o[:, 0] for o in outs]
    has_pos = posc > 0
    num = jnp.where(has_pos, num_m, num_a)
    den = jnp.where(has_pos, cnt_m, jnp.float32(S))
    return jnp.sum(num) / jnp.sum(den)


# speedup vs baseline: 8.0519x; 1.1419x over previous
import functools

import jax
import jax.numpy as jnp
from jax.experimental import pallas as pl
from jax.experimental.pallas import tpu as pltpu

_WINDOW = 100
_WLEN = 2 * _WINDOW + 1


def _focal_body(alpha_ref, x_ref, t_ref, hp_ref, hn_ref,
                num_m_ref, cnt_m_ref, num_a_ref, pos_ref, *, n_j):
    j = pl.program_id(1)
    x0 = x_ref[0]
    x1 = x_ref[1]
    x2 = x_ref[2]
    t = t_ref[...]
    sb = t.shape[1]

    d1 = x1 - x0
    d2 = x2 - x0
    e1 = jnp.exp(d1)
    e2 = jnp.exp(d2)
    se = 1.0 + e1 + e2

    c1 = t == 1
    c2 = t == 2
    dt = jnp.where(c2, d2, jnp.where(c1, d1, 0.0))
    et = jnp.where(c2, e2, jnp.where(c1, e1, 1.0))
    ce = jnp.log(se) - dt
    pt = et * (1.0 / se)

    a0 = alpha_ref[0]
    a1 = alpha_ref[1]
    a2 = alpha_ref[2]
    at = jnp.where(c2, a2, jnp.where(c1, a1, a0))
    om = 1.0 - pt
    focal = at * (om * om) * ce

    pos = jnp.where(t > 0, 1.0, 0.0)
    hp = jnp.where((hp_ref[...] > 0) & (j > 0), 1.0, 0.0)
    hn = jnp.where((hn_ref[...] > 0) & (j < n_j - 1), 1.0, 0.0)
    f = jnp.concatenate([hp, pos, hn], axis=1)
    n = 1
    while n < _WLEN:
        s = min(n, _WLEN - n)
        f = jnp.maximum(f, pltpu.roll(f, f.shape[1] - s, axis=1))
        n += s
    mf = f[:, 128 - _WINDOW:128 - _WINDOW + sb]

    ones = jnp.ones((8, 128), jnp.float32)
    nm = jnp.sum(focal * mf, axis=1, keepdims=True) * ones
    cm = jnp.sum(mf, axis=1, keepdims=True) * ones
    na = jnp.sum(focal, axis=1, keepdims=True) * ones
    pc = jnp.sum(pos, axis=1, keepdims=True) * ones

    @pl.when(j == 0)
    def _():
        num_m_ref[...] = nm
        cnt_m_ref[...] = cm
        num_a_ref[...] = na
        pos_ref[...] = pc

    @pl.when(j > 0)
    def _():
        num_m_ref[...] += nm
        cnt_m_ref[...] += cm
        num_a_ref[...] += na
        pos_ref[...] += pc


@jax.jit
def kernel(inputs, targets, alpha):
    B, S, C = inputs.shape
    x = jnp.transpose(inputs, (2, 0, 1))
    SB = 32768 if S % 32768 == 0 else S
    KB = SB // 128
    NJ = S // SB
    KMAX = S // 128 - 1

    outs = pl.pallas_call(
        functools.partial(_focal_body, n_j=NJ),
        grid=(B // 8, NJ),
        in_specs=[
            pl.BlockSpec(memory_space=pltpu.SMEM),
            pl.BlockSpec((C, 8, SB), lambda i, j: (0, i, j)),
            pl.BlockSpec((8, SB), lambda i, j: (i, j)),
            pl.BlockSpec((8, 128), lambda i, j: (i, jnp.maximum(j * KB - 1, 0))),
            pl.BlockSpec((8, 128), lambda i, j: (i, jnp.minimum((j + 1) * KB, KMAX))),
        ],
        out_specs=[pl.BlockSpec((8, 128), lambda i, j: (i, 0))] * 4,
        out_shape=[jax.ShapeDtypeStruct((B, 128), jnp.float32)] * 4,
        compiler_params=pltpu.CompilerParams(
            dimension_semantics=("parallel", "arbitrary"),
        ),
        name="masked_focal_loss",
    )(alpha, x, targets, targets, targets)

    num_m, cnt_m, num_a, posc = [o[:, 0] for o in outs]
    has_pos = posc > 0
    num = jnp.where(has_pos, num_m, num_a)
    den = jnp.where(has_pos, cnt_m, jnp.float32(S))
    return jnp.sum(num) / jnp.sum(den)


# speedup vs baseline: 8.0895x; 1.0047x over previous
import functools

import jax
import jax.numpy as jnp
from jax.experimental import pallas as pl
from jax.experimental.pallas import tpu as pltpu

_WINDOW = 100
_WLEN = 2 * _WINDOW + 1


def _focal_body(alpha_ref, x_ref, t_ref, hp_ref, hn_ref,
                num_m_ref, cnt_m_ref, num_a_ref, pos_ref, *, n_j):
    j = pl.program_id(1)
    x0 = x_ref[0]
    x1 = x_ref[1]
    x2 = x_ref[2]
    t = t_ref[...]
    sb = t.shape[1]

    d1 = x1 - x0
    d2 = x2 - x0
    e1 = jnp.exp(d1)
    e2 = jnp.exp(d2)
    se = 1.0 + e1 + e2

    c1 = t == 1
    c2 = t == 2
    dt = jnp.where(c2, d2, jnp.where(c1, d1, 0.0))
    et = jnp.where(c2, e2, jnp.where(c1, e1, 1.0))
    ce = jnp.log(se) - dt
    pt = et * (1.0 / se)

    a0 = alpha_ref[0]
    a1 = alpha_ref[1]
    a2 = alpha_ref[2]
    at = jnp.where(c2, a2, jnp.where(c1, a1, a0))
    om = 1.0 - pt
    focal = at * (om * om) * ce

    pos = jnp.where(t > 0, 1.0, 0.0)
    hp = jnp.where((hp_ref[...] > 0) & (j > 0), 1.0, 0.0)
    hn = jnp.where((hn_ref[...] > 0) & (j < n_j - 1), 1.0, 0.0)
    f = jnp.concatenate([hp, pos, hn], axis=1)
    n = 1
    while n < _WLEN:
        s = min(n, _WLEN - n)
        f = jnp.maximum(f, pltpu.roll(f, f.shape[1] - s, axis=1))
        n += s
    mf = f[:, 128 - _WINDOW:128 - _WINDOW + sb]

    ones = jnp.ones((8, 128), jnp.float32)
    nm = jnp.sum(focal * mf, axis=1, keepdims=True) * ones
    cm = jnp.sum(mf, axis=1, keepdims=True) * ones
    na = jnp.sum(focal, axis=1, keepdims=True) * ones
    pc = jnp.sum(pos, axis=1, keepdims=True) * ones

    @pl.when(j == 0)
    def _():
        num_m_ref[...] = nm
        cnt_m_ref[...] = cm
        num_a_ref[...] = na
        pos_ref[...] = pc

    @pl.when(j > 0)
    def _():
        num_m_ref[...] += nm
        cnt_m_ref[...] += cm
        num_a_ref[...] += na
        pos_ref[...] += pc


@jax.jit
def kernel(inputs, targets, alpha):
    B, S, C = inputs.shape
    x = jnp.transpose(inputs, (2, 0, 1))
    SB = 65536 if S % 65536 == 0 else S
    KB = SB // 128
    NJ = S // SB
    KMAX = S // 128 - 1

    outs = pl.pallas_call(
        functools.partial(_focal_body, n_j=NJ),
        grid=(B // 8, NJ),
        in_specs=[
            pl.BlockSpec(memory_space=pltpu.SMEM),
            pl.BlockSpec((C, 8, SB), lambda i, j: (0, i, j)),
            pl.BlockSpec((8, SB), lambda i, j: (i, j)),
            pl.BlockSpec((8, 128), lambda i, j: (i, jnp.maximum(j * KB - 1, 0))),
            pl.BlockSpec((8, 128), lambda i, j: (i, jnp.minimum((j + 1) * KB, KMAX))),
        ],
        out_specs=[pl.BlockSpec((8, 128), lambda i, j: (i, 0))] * 4,
        out_shape=[jax.ShapeDtypeStruct((B, 128), jnp.float32)] * 4,
        compiler_params=pltpu.CompilerParams(
            dimension_semantics=("parallel", "arbitrary"),
        ),
        name="masked_focal_loss",
    )(alpha, x, targets, targets, targets)

    num_m, cnt_m, num_a, posc = [o[:, 0] for o in outs]
    has_pos = posc > 0
    num = jnp.where(has_pos, num_m, num_a)
    den = jnp.where(has_pos, cnt_m, jnp.float32(S))
    return jnp.sum(num) / jnp.sum(den)
